# baseline (speedup 1.0000x reference)
import functools

import jax
import jax.numpy as jnp
from jax import lax
from jax.experimental import pallas as pl
from jax.experimental.pallas import tpu as pltpu
from jax.experimental.pallas import tpu_sc as plsc

CAPACITY = 100000
D_MODEL = 512
BATCH = 4096

TAIL = CAPACITY - BATCH
CH_TAIL = 96
N_TAIL_CHUNKS = TAIL // CH_TAIL
CH_EMB = 64

IMP_PIECE = 12000
IMP_LAST = TAIL - 7 * IMP_PIECE


def _build(nc, ns):
    nw = nc * ns
    emb_chunks_per_w = BATCH // CH_EMB // nw

    mesh = plsc.VectorSubcoreMesh(core_axis_name="c", subcore_axis_name="s")

    @functools.partial(
        pl.kernel,
        mesh=mesh,
        out_type=[
            jax.ShapeDtypeStruct((CAPACITY, D_MODEL), jnp.float32),
            jax.ShapeDtypeStruct((CAPACITY,), jnp.float32),
        ],
        scratch_types=[
            pltpu.VMEM((CH_TAIL, D_MODEL), jnp.float32),
            pltpu.VMEM((IMP_PIECE,), jnp.float32),
        ],
    )
    def k(emb, sig, mem, imp, out_mem, out_imp, buf, ibuf):
        wid = lax.axis_index("s") * nc + lax.axis_index("c")

        for j in range(emb_chunks_per_w):
            off = pl.multiple_of(wid * (emb_chunks_per_w * CH_EMB) + j * CH_EMB, 8)
            pltpu.sync_copy(emb.at[pl.ds(off, CH_EMB)], buf.at[pl.ds(0, CH_EMB)])
            pltpu.sync_copy(buf.at[pl.ds(0, CH_EMB)], out_mem.at[pl.ds(off, CH_EMB)])

        n_w = (N_TAIL_CHUNKS - wid + nw - 1) // nw

        def body(kk, _):
            c = wid + kk * nw
            off = pl.multiple_of(BATCH + c * CH_TAIL, 8)
            pltpu.sync_copy(mem.at[pl.ds(off, CH_TAIL)], buf)
            pltpu.sync_copy(buf, out_mem.at[pl.ds(off, CH_TAIL)])
            return _

        lax.fori_loop(0, n_w, body, 0)

        @pl.when(wid == 0)
        def _():
            pltpu.sync_copy(sig, ibuf.at[pl.ds(0, BATCH)])
            pltpu.sync_copy(ibuf.at[pl.ds(0, BATCH)], out_imp.at[pl.ds(0, BATCH)])

        for p in range(8):
            sz = IMP_LAST if p == 7 else IMP_PIECE
            start = BATCH + p * IMP_PIECE

            @pl.when(wid == p + 1)
            def _(sz=sz, start=start):
                pltpu.sync_copy(imp.at[pl.ds(start, sz)], ibuf.at[pl.ds(0, sz)])
                pltpu.sync_copy(ibuf.at[pl.ds(0, sz)], out_imp.at[pl.ds(start, sz)])

    return k


def kernel(embeddings, loss_signal, memory, importance):
    info = plsc.get_sparse_core_info()
    k = _build(info.num_cores, info.num_subcores)
    out_mem, out_imp = k(embeddings, loss_signal, memory, importance)
    return out_mem, out_imp


# speedup vs baseline: 1.1175x; 1.1175x over previous
import functools

import jax
import jax.numpy as jnp
from jax import lax
from jax.experimental import pallas as pl
from jax.experimental.pallas import tpu as pltpu
from jax.experimental.pallas import tpu_sc as plsc

CAPACITY = 100000
D_MODEL = 512
BATCH = 4096

CH = 32
N_CHUNKS = CAPACITY // CH
EMB_CHUNKS = BATCH // CH
U = 4

TAIL = CAPACITY - BATCH
IMP_PIECE = 12000
IMP_LAST = TAIL - 7 * IMP_PIECE


def _build(nc, ns):
    nw = nc * ns
    steps = -(-N_CHUNKS // nw)
    steps = -(-steps // U) * U
    n_iter = steps // U

    mesh = plsc.VectorSubcoreMesh(core_axis_name="c", subcore_axis_name="s")

    @functools.partial(
        pl.kernel,
        mesh=mesh,
        out_type=[
            jax.ShapeDtypeStruct((CAPACITY, D_MODEL), jnp.float32),
            jax.ShapeDtypeStruct((CAPACITY,), jnp.float32),
        ],
        scratch_types=[
            pltpu.VMEM((U, CH, D_MODEL), jnp.float32),
            pltpu.VMEM((IMP_PIECE,), jnp.float32),
            pltpu.SemaphoreType.DMA((U,)),
            pltpu.SemaphoreType.DMA((U,)),
        ],
    )
    def k(emb, sig, mem, imp, out_mem, out_imp, buf, ibuf, sem_in, sem_out):
        wid = lax.axis_index("s") * nc + lax.axis_index("c")

        @pl.when(wid == 0)
        def _():
            pltpu.sync_copy(sig, ibuf.at[pl.ds(0, BATCH)])
            pltpu.sync_copy(ibuf.at[pl.ds(0, BATCH)], out_imp.at[pl.ds(0, BATCH)])

        for p in range(8):
            sz = IMP_LAST if p == 7 else IMP_PIECE
            start = BATCH + p * IMP_PIECE

            @pl.when(wid == p + 1)
            def _(sz=sz, start=start):
                pltpu.sync_copy(imp.at[pl.ds(start, sz)], ibuf.at[pl.ds(0, sz)])
                pltpu.sync_copy(ibuf.at[pl.ds(0, sz)], out_imp.at[pl.ds(start, sz)])

        def in_copy(c, j):
            off = pl.multiple_of(c * CH, CH)
            dst = buf.at[j]
            below = c < EMB_CHUNKS

            @pl.when(below)
            def _():
                pltpu.make_async_copy(emb.at[pl.ds(off, CH)], dst,
                                      sem_in.at[j]).start()

            @pl.when(jnp.logical_not(below))
            def _():
                pltpu.make_async_copy(mem.at[pl.ds(off, CH)], dst,
                                      sem_in.at[j]).start()

        def wait_in(c, j):
            off = pl.multiple_of(c * CH, CH)
            pltpu.make_async_copy(mem.at[pl.ds(off, CH)], buf.at[j],
                                  sem_in.at[j]).wait()

        def out_copy(c, j):
            off = pl.multiple_of(c * CH, CH)
            return pltpu.make_async_copy(buf.at[j], out_mem.at[pl.ds(off, CH)],
                                         sem_out.at[j])

        def body(i, carry):
            for j in range(U):
                c = wid + (i * U + j) * nw

                @pl.when(c < N_CHUNKS)
                def _(c=c, j=j):
                    @pl.when(i > 0)
                    def _():
                        out_copy(c - U * nw, j).wait()

                    in_copy(c, j)

            for j in range(U):
                c = wid + (i * U + j) * nw

                @pl.when(c < N_CHUNKS)
                def _(c=c, j=j):
                    wait_in(c, j)
                    out_copy(c, j).start()

            return carry

        lax.fori_loop(0, n_iter, body, 0)

        kk_max = (N_CHUNKS - 1 - wid) // nw
        for j in range(U):
            @pl.when(kk_max >= j)
            def _(j=j):
                out_copy(wid, j).wait()

    return k


def kernel(embeddings, loss_signal, memory, importance):
    info = plsc.get_sparse_core_info()
    k = _build(info.num_cores, info.num_subcores)
    out_mem, out_imp = k(embeddings, loss_signal, memory, importance)
    return out_mem, out_imp
